# baseline (speedup 1.0000x reference)
import functools

import jax
import jax.numpy as jnp
from jax import lax
from jax.experimental import pallas as pl
from jax.experimental.pallas import tpu as pltpu
from jax.experimental.pallas import tpu_sc as plsc

NN = 10000
EE = 320000
HH = 128
BB = 128
NB = 10
RB = NN // NB
NC = 2
NS = 16
NWK = NC * NS

KE = 125
CE = EE // KE // NWK
GG = 16
NG = CE // GG
KD = 50
CD = EE // KD // NWK
DROW = 640
DPAD = DROW * NS
RT = 632
NPAD = RT * NS
SCH = 120

_mesh = plsc.VectorSubcoreMesh(core_axis_name="c", subcore_axis_name="s")


@functools.partial(
    pl.kernel,
    out_type=jax.ShapeDtypeStruct((NC, DPAD), jnp.float32),
    mesh=_mesh,
    scratch_types=[
        pltpu.VMEM((CD, KD), jnp.int32),
        pltpu.VMEM((KD,), jnp.float32),
        pltpu.VMEM((DROW,), jnp.float32),
        pltpu.VMEM_SHARED((DPAD,), jnp.float32),
    ],
)
def _deg_call(dst_hbm, ones_hbm, zer_hbm, out_hbm, idx_buf, ones_v, vbuf, dacc):
    c = lax.axis_index("c")
    s = lax.axis_index("s")
    w = s * NC + c
    pltpu.sync_copy(zer_hbm, vbuf)
    pltpu.sync_copy(vbuf, dacc.at[pl.ds(s * DROW, DROW)])
    pltpu.sync_copy(ones_hbm, ones_v)
    pltpu.sync_copy(dst_hbm.at[pl.ds(w * CD, CD)], idx_buf)
    plsc.subcore_barrier()

    def step(j, carry):
        pltpu.sync_copy(ones_v, dacc.at[idx_buf.at[j]], add=True)
        return carry

    lax.fori_loop(0, CD, step, 0)
    plsc.subcore_barrier()
    pltpu.sync_copy(dacc.at[pl.ds(s * DROW, DROW)], vbuf)
    pltpu.sync_copy(vbuf, out_hbm.at[c].at[pl.ds(s * DROW, DROW)])


@functools.partial(
    pl.kernel,
    out_type=jax.ShapeDtypeStruct((NC, NPAD, HH), jnp.float32),
    mesh=_mesh,
    scratch_types=[
        pltpu.VMEM((GG, KE), jnp.int32),
        pltpu.VMEM((GG, KE), jnp.int32),
        pltpu.VMEM((KE, HH), jnp.float32),
        pltpu.VMEM((KE, HH), jnp.float32),
        pltpu.VMEM_SHARED((NPAD, HH), jnp.float32),
        pltpu.SemaphoreType.DMA,
        pltpu.SemaphoreType.DMA,
    ],
)
def _edge_call(z_hbm, src_hbm, dst_hbm, zer_hbm, out_hbm,
               sbuf, dbuf, gb0, gb1, acc, sem0, sem1):
    c = lax.axis_index("c")
    s = lax.axis_index("s")
    w = s * NC + c
    gbufs = (gb0, gb1)
    sems = (sem0, sem1)

    pltpu.sync_copy(zer_hbm, gb0)
    for k in range(5):
        pltpu.async_copy(gb0.at[pl.ds(0, SCH)],
                         acc.at[pl.ds(s * RT + k * SCH, SCH)], sem0)
    pltpu.async_copy(gb0.at[pl.ds(0, RT - 5 * SCH)],
                     acc.at[pl.ds(s * RT + 5 * SCH, RT - 5 * SCH)], sem1)
    for k in range(5):
        pltpu.make_async_copy(gb0.at[pl.ds(0, SCH)],
                              acc.at[pl.ds(s * RT + k * SCH, SCH)],
                              sem0).wait()
    pltpu.make_async_copy(gb0.at[pl.ds(0, RT - 5 * SCH)],
                          acc.at[pl.ds(s * RT + 5 * SCH, RT - 5 * SCH)],
                          sem1).wait()
    plsc.subcore_barrier()

    def group(g, carry):
        base = w * CE + g * GG
        pltpu.sync_copy(src_hbm.at[pl.ds(base, GG)], sbuf)
        pltpu.sync_copy(dst_hbm.at[pl.ds(base, GG)], dbuf)
        pltpu.async_copy(z_hbm.at[sbuf.at[0]], gb0, sem0)

        def step(i, carry2):
            j0 = i * 2
            for b in range(2):
                j = j0 + b

                @pl.when(j + 1 < GG)
                def _():
                    pltpu.async_copy(z_hbm.at[sbuf.at[j + 1]],
                                     gbufs[1 - b], sems[1 - b])

                pltpu.make_async_copy(z_hbm.at[sbuf.at[j]],
                                      gbufs[b], sems[b]).wait()
                pltpu.sync_copy(gbufs[b], acc.at[dbuf.at[j]], add=True)
            return carry2

        lax.fori_loop(0, GG // 2, step, 0)
        return carry

    lax.fori_loop(0, NG, group, 0)
    plsc.subcore_barrier()

    def _ochunk(k):
        size = SCH if k < 5 else RT - 5 * SCH
        off = s * RT + k * SCH
        b = k & 1
        return size, off, gbufs[b], sems[b]

    for k in range(6):
        size, off, gb, sem = _ochunk(k)
        if k >= 2:
            psize, poff, pgb, psem = _ochunk(k - 2)
            pltpu.make_async_copy(pgb.at[pl.ds(0, psize)],
                                  out_hbm.at[c].at[pl.ds(poff, psize)],
                                  psem).wait()
        pltpu.sync_copy(acc.at[pl.ds(off, size)], gb.at[pl.ds(0, size)])
        pltpu.async_copy(gb.at[pl.ds(0, size)],
                         out_hbm.at[c].at[pl.ds(off, size)], sem)
    for k in (4, 5):
        size, off, gb, sem = _ochunk(k)
        pltpu.make_async_copy(gb.at[pl.ds(0, size)],
                              out_hbm.at[c].at[pl.ds(off, size)], sem).wait()


def _tca_body(d0_ref, d1_ref, x_ref, w1_ref, z1_ref, dinv_ref):
    dinv = lax.rsqrt(d0_ref[0] + d1_ref[0] + 1.0)
    z1_ref[...] = jnp.dot(x_ref[...], w1_ref[...],
                          preferred_element_type=jnp.float32) * dinv
    dinv_ref[0] = dinv


def _tcb_body(acc_ref, z1_ref, dinv_ref, b1_ref, w2_ref, z2_ref):
    dinv = dinv_ref[0]
    t = acc_ref[0] + acc_ref[1] + z1_ref[...]
    h1 = jnp.maximum(t * dinv + b1_ref[...], 0.0)
    z2_ref[...] = jnp.dot(h1, w2_ref[...],
                          preferred_element_type=jnp.float32) * dinv


def _tcc_body(acc_ref, z2_ref, dinv_ref, b2_ref, batch_ref,
              wa_ref, wc_ref, wn_ref, wt_ref,
              ba_ref, bc_ref, bn_ref, bt_ref,
              ya_ref, yc_ref, yn_ref, yt_ref, summed, cnt):
    i = pl.program_id(0)

    @pl.when(i == 0)
    def _():
        summed[...] = jnp.zeros_like(summed)
        cnt[...] = jnp.zeros_like(cnt)

    dinv = dinv_ref[0]
    t = acc_ref[0] + acc_ref[1] + z2_ref[...]
    h2 = jnp.maximum(t * dinv + b2_ref[...], 0.0)
    bb = batch_ref[0]
    seg = lax.broadcasted_iota(jnp.int32, (BB, 1), 0)
    m = (bb == seg).astype(jnp.float32)
    summed[...] += jnp.dot(m, h2, preferred_element_type=jnp.float32)
    cnt[...] += jnp.sum(m, axis=1, keepdims=True)

    @pl.when(i == NB - 1)
    def _():
        pool = summed[...] / jnp.maximum(cnt[...], 1.0)
        ya_ref[...] = jnp.dot(pool, wa_ref[...],
                              preferred_element_type=jnp.float32) + ba_ref[...]
        yc_ref[...] = jnp.dot(pool, wc_ref[...],
                              preferred_element_type=jnp.float32) + bc_ref[...]
        yn_ref[...] = jnp.dot(pool, wn_ref[...],
                              preferred_element_type=jnp.float32) + bn_ref[...]
        yt_ref[...] = jnp.dot(pool, wt_ref[...],
                              preferred_element_type=jnp.float32) + bt_ref[...]


def _whole(shape):
    return pl.BlockSpec(shape, lambda i: tuple(0 for _ in shape))


_tca = pl.pallas_call(
    _tca_body,
    grid=(NB,),
    in_specs=[
        pl.BlockSpec((1, RB, 1), lambda i: (i, 0, 0)),
        pl.BlockSpec((1, RB, 1), lambda i: (i, 0, 0)),
        pl.BlockSpec((RB, HH), lambda i: (i, 0)),
        _whole((HH, HH)),
    ],
    out_specs=[
        pl.BlockSpec((RB, HH), lambda i: (i, 0)),
        pl.BlockSpec((1, RB, 1), lambda i: (i, 0, 0)),
    ],
    out_shape=[
        jax.ShapeDtypeStruct((NN, HH), jnp.float32),
        jax.ShapeDtypeStruct((NB, RB, 1), jnp.float32),
    ],
)

_tcb = pl.pallas_call(
    _tcb_body,
    grid=(NB,),
    in_specs=[
        pl.BlockSpec((NC, RB, HH), lambda i: (0, i, 0)),
        pl.BlockSpec((RB, HH), lambda i: (i, 0)),
        pl.BlockSpec((1, RB, 1), lambda i: (i, 0, 0)),
        _whole((1, HH)),
        _whole((HH, HH)),
    ],
    out_specs=pl.BlockSpec((RB, HH), lambda i: (i, 0)),
    out_shape=jax.ShapeDtypeStruct((NN, HH), jnp.float32),
)

_tcc = pl.pallas_call(
    _tcc_body,
    grid=(NB,),
    in_specs=[
        pl.BlockSpec((NC, RB, HH), lambda i: (0, i, 0)),
        pl.BlockSpec((RB, HH), lambda i: (i, 0)),
        pl.BlockSpec((1, RB, 1), lambda i: (i, 0, 0)),
        _whole((1, HH)),
        pl.BlockSpec((1, 1, RB), lambda i: (i, 0, 0)),
        _whole((HH, 4)), _whole((HH, 81)), _whole((HH, 9)), _whole((HH, 16)),
        _whole((1, 4)), _whole((1, 81)), _whole((1, 9)), _whole((1, 16)),
    ],
    out_specs=[_whole((BB, 4)), _whole((BB, 81)),
               _whole((BB, 9)), _whole((BB, 16))],
    out_shape=[
        jax.ShapeDtypeStruct((BB, 4), jnp.float32),
        jax.ShapeDtypeStruct((BB, 81), jnp.float32),
        jax.ShapeDtypeStruct((BB, 9), jnp.float32),
        jax.ShapeDtypeStruct((BB, 16), jnp.float32),
    ],
    scratch_shapes=[
        pltpu.VMEM((BB, HH), jnp.float32),
        pltpu.VMEM((BB, 1), jnp.float32),
    ],
)


def kernel(x, edge_index, batch, W1, b1, W2, b2, Wa, ba, Wc, bc, Wn, bn, Wt, bt):
    src = edge_index[0]
    dst = edge_index[1]
    dst_kd = dst.reshape(EE // KD, KD)
    src_ke = src.reshape(EE // KE, KE)
    dst_ke = dst.reshape(EE // KE, KE)
    ones_kd = jnp.ones((KD,), jnp.float32)
    zer_d = jnp.zeros((DROW,), jnp.float32)
    zer_r = jnp.zeros((KE, HH), jnp.float32)

    degp = _deg_call(dst_kd, ones_kd, zer_d)
    d0 = degp[0, :NN].reshape(NB, RB, 1)
    d1 = degp[1, :NN].reshape(NB, RB, 1)
    z1, dinv3 = _tca(d0, d1, x, W1)
    acc1 = _edge_call(z1, src_ke, dst_ke, zer_r)
    z2 = _tcb(acc1, z1, dinv3, b1.reshape(1, HH), W2)
    acc2 = _edge_call(z2, src_ke, dst_ke, zer_r)
    batch3 = batch.reshape(NB, 1, RB)
    return _tcc(acc2, z2, dinv3, b2.reshape(1, HH), batch3,
                Wa, Wc, Wn, Wt,
                ba.reshape(1, 4), bc.reshape(1, 81),
                bn.reshape(1, 9), bt.reshape(1, 16))


# speedup vs baseline: 1.0385x; 1.0385x over previous
import functools

import jax
import jax.numpy as jnp
from jax import lax
from jax.experimental import pallas as pl
from jax.experimental.pallas import tpu as pltpu
from jax.experimental.pallas import tpu_sc as plsc

NN = 10000
EE = 320000
HH = 128
BB = 128
NB = 10
RB = NN // NB
NC = 2
NS = 16
NWK = NC * NS

KE = 125
CE = EE // KE // NWK
GG = 16
NG = CE // GG
DROW = 640
DPAD = DROW * NS
RT = 632
NPAD = RT * NS
SCH = 120

_mesh = plsc.VectorSubcoreMesh(core_axis_name="c", subcore_axis_name="s")


@functools.partial(
    pl.kernel,
    out_type=jax.ShapeDtypeStruct((NC, DPAD), jnp.float32),
    mesh=_mesh,
    scratch_types=[
        pltpu.VMEM((CE, KE), jnp.int32),
        pltpu.VMEM((KE,), jnp.float32),
        pltpu.VMEM((DROW,), jnp.float32),
        pltpu.VMEM_SHARED((DPAD,), jnp.float32),
    ],
)
def _deg_call(dst_hbm, ones_hbm, zer_hbm, out_hbm, idx_buf, ones_v, vbuf, dacc):
    c = lax.axis_index("c")
    s = lax.axis_index("s")
    w = s * NC + c
    pltpu.sync_copy(zer_hbm, vbuf)
    pltpu.sync_copy(vbuf, dacc.at[pl.ds(s * DROW, DROW)])
    pltpu.sync_copy(ones_hbm, ones_v)
    pltpu.sync_copy(dst_hbm.at[pl.ds(w * CE, CE)], idx_buf)
    plsc.subcore_barrier()

    def step(j, carry):
        pltpu.sync_copy(ones_v, dacc.at[idx_buf.at[j]], add=True)
        return carry

    lax.fori_loop(0, CE, step, 0)
    plsc.subcore_barrier()
    pltpu.sync_copy(dacc.at[pl.ds(s * DROW, DROW)], vbuf)
    pltpu.sync_copy(vbuf, out_hbm.at[c].at[pl.ds(s * DROW, DROW)])


@functools.partial(
    pl.kernel,
    out_type=jax.ShapeDtypeStruct((NC, NPAD, HH), jnp.float32),
    mesh=_mesh,
    scratch_types=[
        pltpu.VMEM((GG, KE), jnp.int32),
        pltpu.VMEM((GG, KE), jnp.int32),
        pltpu.VMEM((KE, HH), jnp.float32),
        pltpu.VMEM((KE, HH), jnp.float32),
        pltpu.VMEM_SHARED((NPAD, HH), jnp.float32),
        pltpu.SemaphoreType.DMA,
        pltpu.SemaphoreType.DMA,
    ],
)
def _edge_call(z_hbm, src_hbm, dst_hbm, zer_hbm, out_hbm,
               sbuf, dbuf, gb0, gb1, acc, sem0, sem1):
    c = lax.axis_index("c")
    s = lax.axis_index("s")
    w = s * NC + c
    gbufs = (gb0, gb1)
    sems = (sem0, sem1)

    pltpu.sync_copy(zer_hbm, gb0)
    for k in range(5):
        pltpu.async_copy(gb0.at[pl.ds(0, SCH)],
                         acc.at[pl.ds(s * RT + k * SCH, SCH)], sem0)
    pltpu.async_copy(gb0.at[pl.ds(0, RT - 5 * SCH)],
                     acc.at[pl.ds(s * RT + 5 * SCH, RT - 5 * SCH)], sem1)
    for k in range(5):
        pltpu.make_async_copy(gb0.at[pl.ds(0, SCH)],
                              acc.at[pl.ds(s * RT + k * SCH, SCH)],
                              sem0).wait()
    pltpu.make_async_copy(gb0.at[pl.ds(0, RT - 5 * SCH)],
                          acc.at[pl.ds(s * RT + 5 * SCH, RT - 5 * SCH)],
                          sem1).wait()
    plsc.subcore_barrier()

    def group(g, carry):
        base = w * CE + g * GG
        pltpu.sync_copy(src_hbm.at[pl.ds(base, GG)], sbuf)
        pltpu.sync_copy(dst_hbm.at[pl.ds(base, GG)], dbuf)
        pltpu.async_copy(z_hbm.at[sbuf.at[0]], gb0, sem0)

        def step(i, carry2):
            j0 = i * 2
            for b in range(2):
                j = j0 + b

                @pl.when(j + 1 < GG)
                def _():
                    pltpu.async_copy(z_hbm.at[sbuf.at[j + 1]],
                                     gbufs[1 - b], sems[1 - b])

                pltpu.make_async_copy(z_hbm.at[sbuf.at[j]],
                                      gbufs[b], sems[b]).wait()
                pltpu.sync_copy(gbufs[b], acc.at[dbuf.at[j]], add=True)
            return carry2

        lax.fori_loop(0, GG // 2, step, 0)
        return carry

    lax.fori_loop(0, NG, group, 0)
    plsc.subcore_barrier()

    def _ochunk(k):
        size = SCH if k < 5 else RT - 5 * SCH
        off = s * RT + k * SCH
        b = k & 1
        return size, off, gbufs[b], sems[b]

    for k in range(6):
        size, off, gb, sem = _ochunk(k)
        if k >= 2:
            psize, poff, pgb, psem = _ochunk(k - 2)
            pltpu.make_async_copy(pgb.at[pl.ds(0, psize)],
                                  out_hbm.at[c].at[pl.ds(poff, psize)],
                                  psem).wait()
        pltpu.sync_copy(acc.at[pl.ds(off, size)], gb.at[pl.ds(0, size)])
        pltpu.async_copy(gb.at[pl.ds(0, size)],
                         out_hbm.at[c].at[pl.ds(off, size)], sem)
    for k in (4, 5):
        size, off, gb, sem = _ochunk(k)
        pltpu.make_async_copy(gb.at[pl.ds(0, size)],
                              out_hbm.at[c].at[pl.ds(off, size)], sem).wait()


def _tca_body(d0_ref, d1_ref, x_ref, w1_ref, z1_ref, dinv_ref):
    dinv = lax.rsqrt(d0_ref[0] + d1_ref[0] + 1.0)
    z1_ref[...] = jnp.dot(x_ref[...], w1_ref[...],
                          preferred_element_type=jnp.float32) * dinv
    dinv_ref[0] = dinv


def _tcb_body(acc_ref, z1_ref, dinv_ref, b1_ref, w2_ref, z2_ref):
    dinv = dinv_ref[0]
    t = acc_ref[0] + acc_ref[1] + z1_ref[...]
    h1 = jnp.maximum(t * dinv + b1_ref[...], 0.0)
    z2_ref[...] = jnp.dot(h1, w2_ref[...],
                          preferred_element_type=jnp.float32) * dinv


def _tcc_body(acc_ref, z2_ref, dinv_ref, b2_ref, batch_ref,
              wa_ref, wc_ref, wn_ref, wt_ref,
              ba_ref, bc_ref, bn_ref, bt_ref,
              ya_ref, yc_ref, yn_ref, yt_ref, summed, cnt):
    i = pl.program_id(0)

    @pl.when(i == 0)
    def _():
        summed[...] = jnp.zeros_like(summed)
        cnt[...] = jnp.zeros_like(cnt)

    dinv = dinv_ref[0]
    t = acc_ref[0] + acc_ref[1] + z2_ref[...]
    h2 = jnp.maximum(t * dinv + b2_ref[...], 0.0)
    bb = batch_ref[0]
    seg = lax.broadcasted_iota(jnp.int32, (BB, 1), 0)
    m = (bb == seg).astype(jnp.float32)
    summed[...] += jnp.dot(m, h2, preferred_element_type=jnp.float32)
    cnt[...] += jnp.sum(m, axis=1, keepdims=True)

    @pl.when(i == NB - 1)
    def _():
        pool = summed[...] / jnp.maximum(cnt[...], 1.0)
        ya_ref[...] = jnp.dot(pool, wa_ref[...],
                              preferred_element_type=jnp.float32) + ba_ref[...]
        yc_ref[...] = jnp.dot(pool, wc_ref[...],
                              preferred_element_type=jnp.float32) + bc_ref[...]
        yn_ref[...] = jnp.dot(pool, wn_ref[...],
                              preferred_element_type=jnp.float32) + bn_ref[...]
        yt_ref[...] = jnp.dot(pool, wt_ref[...],
                              preferred_element_type=jnp.float32) + bt_ref[...]


def _whole(shape):
    return pl.BlockSpec(shape, lambda i: tuple(0 for _ in shape))


_tca = pl.pallas_call(
    _tca_body,
    grid=(NB,),
    in_specs=[
        pl.BlockSpec((1, RB, 1), lambda i: (i, 0, 0)),
        pl.BlockSpec((1, RB, 1), lambda i: (i, 0, 0)),
        pl.BlockSpec((RB, HH), lambda i: (i, 0)),
        _whole((HH, HH)),
    ],
    out_specs=[
        pl.BlockSpec((RB, HH), lambda i: (i, 0)),
        pl.BlockSpec((1, RB, 1), lambda i: (i, 0, 0)),
    ],
    out_shape=[
        jax.ShapeDtypeStruct((NN, HH), jnp.float32),
        jax.ShapeDtypeStruct((NB, RB, 1), jnp.float32),
    ],
)

_tcb = pl.pallas_call(
    _tcb_body,
    grid=(NB,),
    in_specs=[
        pl.BlockSpec((NC, RB, HH), lambda i: (0, i, 0)),
        pl.BlockSpec((RB, HH), lambda i: (i, 0)),
        pl.BlockSpec((1, RB, 1), lambda i: (i, 0, 0)),
        _whole((1, HH)),
        _whole((HH, HH)),
    ],
    out_specs=pl.BlockSpec((RB, HH), lambda i: (i, 0)),
    out_shape=jax.ShapeDtypeStruct((NN, HH), jnp.float32),
)

_tcc = pl.pallas_call(
    _tcc_body,
    grid=(NB,),
    in_specs=[
        pl.BlockSpec((NC, RB, HH), lambda i: (0, i, 0)),
        pl.BlockSpec((RB, HH), lambda i: (i, 0)),
        pl.BlockSpec((1, RB, 1), lambda i: (i, 0, 0)),
        _whole((1, HH)),
        pl.BlockSpec((1, 1, RB), lambda i: (i, 0, 0)),
        _whole((HH, 4)), _whole((HH, 81)), _whole((HH, 9)), _whole((HH, 16)),
        _whole((1, 4)), _whole((1, 81)), _whole((1, 9)), _whole((1, 16)),
    ],
    out_specs=[_whole((BB, 4)), _whole((BB, 81)),
               _whole((BB, 9)), _whole((BB, 16))],
    out_shape=[
        jax.ShapeDtypeStruct((BB, 4), jnp.float32),
        jax.ShapeDtypeStruct((BB, 81), jnp.float32),
        jax.ShapeDtypeStruct((BB, 9), jnp.float32),
        jax.ShapeDtypeStruct((BB, 16), jnp.float32),
    ],
    scratch_shapes=[
        pltpu.VMEM((BB, HH), jnp.float32),
        pltpu.VMEM((BB, 1), jnp.float32),
    ],
)


def kernel(x, edge_index, batch, W1, b1, W2, b2, Wa, ba, Wc, bc, Wn, bn, Wt, bt):
    e3 = edge_index.reshape(2, EE // KE, KE)
    src_ke = e3[0]
    dst_ke = e3[1]
    ones_ke = jnp.ones((KE,), jnp.float32)
    zer_d = jnp.zeros((DROW,), jnp.float32)
    zer_r = jnp.zeros((KE, HH), jnp.float32)

    degp = _deg_call(dst_ke, ones_ke, zer_d)
    d0 = degp[0, :NN].reshape(NB, RB, 1)
    d1 = degp[1, :NN].reshape(NB, RB, 1)
    z1, dinv3 = _tca(d0, d1, x, W1)
    acc1 = _edge_call(z1, src_ke, dst_ke, zer_r)
    z2 = _tcb(acc1, z1, dinv3, b1.reshape(1, HH), W2)
    acc2 = _edge_call(z2, src_ke, dst_ke, zer_r)
    batch3 = batch.reshape(NB, 1, RB)
    return _tcc(acc2, z2, dinv3, b2.reshape(1, HH), batch3,
                Wa, Wc, Wn, Wt,
                ba.reshape(1, 4), bc.reshape(1, 81),
                bn.reshape(1, 9), bt.reshape(1, 16))
